# baseline (speedup 1.0000x reference)
import functools

import jax
import jax.numpy as jnp
from jax import lax
from jax.experimental import pallas as pl
from jax.experimental.pallas import tpu as pltpu
from jax.experimental.pallas import tpu_sc as plsc

B, H, S, D, K = 16, 12, 577, 768, 64
NEG = -1e30
BIG = 1 << 30


SP = 640
CH = SP // 128


def _topk_body(cls_ref, idx_ref):
    ss = []
    for c in range(CH):
        x = cls_ref[0, :, c * 128:(c + 1) * 128]
        for h in range(1, H):
            x = x + cls_ref[h, :, c * 128:(c + 1) * 128]
        ss.append(x * (1.0 / 12.0))
    pos0 = lax.broadcasted_iota(jnp.int32, (B, 128), 1)
    ss[0] = jnp.where(pos0 == 0, NEG, ss[0])
    poss = [pos0 + 128 * c for c in range(CH)]
    kio = lax.broadcasted_iota(jnp.int32, (B, K), 1)
    base = lax.broadcasted_iota(jnp.int32, (B, K), 0)

    def _lexmax(m, pi, m2, p2):
        take = (m2 > m) | ((m2 == m) & (p2 < pi))
        return jnp.where(take, m2, m), jnp.where(take, p2, pi)

    def step(r, carry):
        *sch, iv = carry
        m, pi = sch[0], poss[0]
        for c in range(1, CH):
            m, pi = _lexmax(m, pi, sch[c], poss[c])
        sh = 1
        while sh < 128:
            m, pi = _lexmax(m, pi, jnp.roll(m, sh, 1), jnp.roll(pi, sh, 1))
            sh *= 2
        iv = jnp.where(kio == r, pi[:, :K] * B + base, iv)
        sch = [jnp.where(poss[c] == pi, NEG, sch[c]) for c in range(CH)]
        return (*sch, iv)

    carry = lax.fori_loop(0, K, step,
                          (*ss, jnp.zeros((B, K), jnp.int32)))
    idx_ref[:, :] = carry[-1]


def _topk_call(cls_t, *, interpret=False):
    return pl.pallas_call(
        _topk_body,
        out_shape=jax.ShapeDtypeStruct((B, K), jnp.int32),
        interpret=interpret,
    )(cls_t)


@functools.lru_cache(maxsize=None)
def _gather_call():
    info = plsc.get_sparse_core_info()
    NC, NS = info.num_cores, info.num_subcores
    NW = NC * NS
    bpw = (B * K) // NW
    hpb = K // bpw
    mesh = plsc.VectorSubcoreMesh(
        core_axis_name="c", subcore_axis_name="s", num_cores=NC)

    @functools.partial(
        pl.kernel,
        out_type=jax.ShapeDtypeStruct((B, K, D), jnp.float32),
        mesh=mesh,
        scratch_types=[
            pltpu.VMEM((bpw,), jnp.int32),
            pltpu.VMEM((bpw, D), jnp.float32),
            pltpu.SemaphoreType.DMA,
        ],
        compiler_params=pltpu.CompilerParams(use_tc_tiling_on_sc=True),
    )
    def gk(table_hbm, idx_hbm, out_hbm, idx_v, rows_v, sem):
        wid = lax.axis_index("s") * NC + lax.axis_index("c")
        b = wid // hpb
        off = (wid % hpb) * bpw
        pltpu.sync_copy(idx_hbm.at[b, pl.ds(off, bpw)], idx_v)
        pltpu.async_copy(table_hbm.at[idx_v], rows_v, sem).wait()
        pltpu.sync_copy(rows_v, out_hbm.at[b, pl.ds(off, bpw)])

    return gk


def kernel(attn_last, hidden_states):
    cls_t = attn_last[:, :, 0, :].transpose(1, 0, 2)
    cls_t = jnp.pad(cls_t, ((0, 0), (0, 0), (0, SP - S)),
                    constant_values=NEG)
    table = hidden_states.transpose(1, 0, 2).reshape(S * B, D)
    idx = _topk_call(cls_t)
    return _gather_call()(table, idx)
